# Initial kernel scaffold; baseline (speedup 1.0000x reference)
import functools

import jax
import jax.numpy as jnp
from jax import lax
from jax.experimental import pallas as pl
from jax.experimental.pallas import tpu as pltpu
from jax.experimental.pallas import tpu_sc as plsc

SR = 137.8
P = 128
B = 4
N = 512
T = 4134
SIGMA = 3.6 / 1000.0 * SR
INV_SIG = 1.0 / SIGMA
W = 6
T_CAP = 4134.0

NC = 2
NS = 16
L = 16
NPG = 8
NP = N + L

NCH = 4
TT = 1040
BUFC = TT + L


def _render_sc(sf, ef, vel, pit):
    mesh = plsc.VectorSubcoreMesh(core_axis_name="c", subcore_axis_name="s")

    @functools.partial(
        pl.kernel,
        mesh=mesh,
        out_type=jax.ShapeDtypeStruct((B, 3 * P, T), jnp.float32),
        scratch_types=[
            pltpu.VMEM((NP,), jnp.float32),
            pltpu.VMEM((NP,), jnp.float32),
            pltpu.VMEM((NP,), jnp.float32),
            pltpu.VMEM((NP,), jnp.int32),
            pltpu.VMEM((NP,), jnp.int32),
            pltpu.VMEM((48, BUFC), jnp.float32),
        ],
    )
    def k(sf_hbm, ef_hbm, vel_hbm, pit_hbm, out_hbm, sfv, efv, velv, pitv, wl, buf):
        cid = lax.axis_index("c")
        sid = lax.axis_index("s")
        wid = sid * NC + cid
        b = wid // NPG
        pg = wid % NPG
        plo = pg * 16

        pltpu.sync_copy(sf_hbm.at[b], sfv)
        pltpu.sync_copy(ef_hbm.at[b], efv)
        pltpu.sync_copy(vel_hbm.at[b], velv)
        pltpu.sync_copy(pit_hbm.at[b], pitv)

        iota = lax.broadcasted_iota(jnp.int32, (L,), 0)

        def init_wl(i, _):
            wl[pl.ds(i * L, L)] = jnp.full((L,), N, jnp.int32)
            return 0
        lax.fori_loop(0, NP // L, init_wl, 0)

        def route(i, cnt):
            p16 = pitv[pl.ds(i * L, L)]
            sel = (p16 >= plo) & (p16 < plo + 16)
            plsc.store_compressed(wl.at[pl.ds(cnt, L)], i * L + iota, sel)
            return cnt + plsc.all_reduce_population_count(sel)
        cnt = lax.fori_loop(0, N // L, route, 0)
        ngrp = (cnt + (L - 1)) // L

        for ch in range(NCH):
            c0 = ch * TT
            cw = min(TT, T - c0)

            def zrow(r, _):
                def zcol(j, _):
                    buf[r, pl.ds(j * L, L)] = jnp.zeros((L,), jnp.float32)
                    return 0
                lax.fori_loop(0, BUFC // L, zcol, 0)
                return 0
            lax.fori_loop(0, 48, zrow, 0)

            def grp_body(g, _):
                ids = wl[pl.ds(g * L, L)]
                sfg = plsc.load_gather(sfv, [ids])
                efg = plsc.load_gather(efv, [ids])
                velg = plsc.load_gather(velv, [ids])
                pitg = plsc.load_gather(pitv, [ids])

                s0i = sfg.astype(jnp.int32)
                cfi = efg.astype(jnp.int32)
                ceii = cfi + jnp.where(cfi.astype(jnp.float32) < efg, 1, 0)
                s0g = s0i.astype(jnp.float32)
                e0g = jnp.minimum(ceii.astype(jnp.float32), T_CAP)
                lo16 = jnp.maximum(jnp.maximum(s0i - W, 0), c0)
                hi16 = jnp.minimum(jnp.minimum(ceii + (W + 1), T), c0 + cw)
                row16 = pitg - plo

                for j in range(L):
                    lane = jnp.full((L,), j, jnp.int32)
                    m = iota == j
                    sfb = jnp.take(sfg, lane, mode="promise_in_bounds")
                    efb = jnp.take(efg, lane, mode="promise_in_bounds")
                    velb = jnp.take(velg, lane, mode="promise_in_bounds")
                    s0b = jnp.take(s0g, lane, mode="promise_in_bounds")
                    e0b = jnp.take(e0g, lane, mode="promise_in_bounds")
                    loj = jnp.sum(jnp.where(m, lo16, 0))
                    hij = jnp.sum(jnp.where(m, hi16, 0))
                    rowj = jnp.sum(jnp.where(m, row16, 0))
                    ntrip = (jnp.maximum(hij - loj, 0) + (L - 1)) // L

                    def frame_body(kk, _):
                        t0 = loj + kk * L
                        tvi = t0 + iota
                        tvf = tvi.astype(jnp.float32)
                        valid = tvi < hij
                        zon = (tvf - sfb) * INV_SIG
                        on = jnp.exp(-0.5 * zon * zon) * velb
                        box = jnp.where((tvf >= s0b) & (tvf < e0b), 1.0, 0.0)
                        zof = (tvf - efb) * INV_SIG
                        g10 = jnp.where(tvf >= efb,
                                        jnp.exp(-0.5 * zof * zof) * 0.1, 0.0)
                        sus = box + g10
                        zero = jnp.zeros((L,), jnp.float32)
                        on = jnp.where(valid, on, zero)
                        sus = jnp.where(valid, sus, zero)
                        velc = sus * velb
                        col = t0 - c0
                        plsc.addupdate(buf.at[rowj, pl.ds(col, L)], on)
                        plsc.addupdate(buf.at[rowj + 16, pl.ds(col, L)], sus)
                        plsc.addupdate(buf.at[rowj + 32, pl.ds(col, L)], velc)
                        return 0

                    lax.fori_loop(0, ntrip, frame_body, 0)
                return 0
            lax.fori_loop(0, ngrp, grp_body, 0)

            def crow(r, _):
                def ccol(j, _):
                    v = buf[r, pl.ds(j * L, L)]
                    buf[r, pl.ds(j * L, L)] = jnp.clip(v, 0.0, 1.0)
                    return 0
                lax.fori_loop(0, BUFC // L, ccol, 0)
                return 0
            lax.fori_loop(0, 48, crow, 0)

            pltpu.sync_copy(buf.at[pl.ds(0, 16), pl.ds(0, cw)],
                            out_hbm.at[b, pl.ds(plo, 16), pl.ds(c0, cw)])
            pltpu.sync_copy(buf.at[pl.ds(16, 16), pl.ds(0, cw)],
                            out_hbm.at[b, pl.ds(P + plo, 16), pl.ds(c0, cw)])
            pltpu.sync_copy(buf.at[pl.ds(32, 16), pl.ds(0, cw)],
                            out_hbm.at[b, pl.ds(2 * P + plo, 16), pl.ds(c0, cw)])

    return k(sf, ef, vel, pit)


def kernel(note_start, note_end, note_vel, note_pitch, dur_sec):
    sr = jnp.float32(SR)
    sf = (note_start * sr).astype(jnp.float32)
    ef = (note_end * sr).astype(jnp.float32)
    vel = note_vel.astype(jnp.float32)
    pit = jnp.clip(note_pitch.astype(jnp.int32), 0, P - 1)
    pad = ((0, 0), (0, L))
    sf = jnp.pad(sf, pad, constant_values=-1e6)
    ef = jnp.pad(ef, pad, constant_values=-1e6)
    vel = jnp.pad(vel, pad, constant_values=0.0)
    pit = jnp.pad(pit, pad, constant_values=-1)
    return _render_sc(sf, ef, vel, pit)


# speedup vs baseline: 1.0069x; 1.0069x over previous
import functools

import jax
import jax.numpy as jnp
from jax import lax
from jax.experimental import pallas as pl
from jax.experimental.pallas import tpu as pltpu
from jax.experimental.pallas import tpu_sc as plsc

SR = 137.8
P = 128
B = 4
N = 512
T = 4134
SIGMA = 3.6 / 1000.0 * SR
INV_SIG = 1.0 / SIGMA
W = 6
T_CAP = 4134.0
TPAD = 4160

NC = 2
NS = 16
L = 16
NPG = 8
NP = N + L

NCH = 4
TT = 1040
BUFC = TT + L


def _take16(x, idx):
    dnums = lax.GatherDimensionNumbers(
        offset_dims=(), collapsed_slice_dims=(0,), start_index_map=(0,))
    return lax.gather(x, idx[:, None], dnums, (1,),
                      mode=lax.GatherScatterMode.PROMISE_IN_BOUNDS)


def _render_sc(sf, ef, vel, pit):
    mesh = plsc.VectorSubcoreMesh(core_axis_name="c", subcore_axis_name="s")

    @functools.partial(
        pl.kernel,
        mesh=mesh,
        out_type=jax.ShapeDtypeStruct((B, 3 * P, TPAD), jnp.float32),
        scratch_types=[
            pltpu.VMEM((NP,), jnp.float32),
            pltpu.VMEM((NP,), jnp.float32),
            pltpu.VMEM((NP,), jnp.float32),
            pltpu.VMEM((NP,), jnp.int32),
            pltpu.VMEM((NP,), jnp.int32),
            pltpu.VMEM((48, BUFC), jnp.float32),
        ],
        compiler_params=pltpu.CompilerParams(use_tc_tiling_on_sc=False,
                                             needs_layout_passes=False),
    )
    def k(sf_hbm, ef_hbm, vel_hbm, pit_hbm, out_hbm, sfv, efv, velv, pitv, wl, buf):
        cid = lax.axis_index("c")
        sid = lax.axis_index("s")
        wid = sid * NC + cid
        b = wid // NPG
        pg = wid % NPG
        plo = pg * 16

        pltpu.sync_copy(sf_hbm.at[b], sfv)
        pltpu.sync_copy(ef_hbm.at[b], efv)
        pltpu.sync_copy(vel_hbm.at[b], velv)
        pltpu.sync_copy(pit_hbm.at[b], pitv)

        iota = lax.broadcasted_iota(jnp.int32, (L,), 0)

        def init_wl(i, _):
            wl[pl.ds(i * L, L)] = jnp.full((L,), N, jnp.int32)
            return 0
        lax.fori_loop(0, NP // L, init_wl, 0)

        def route(i, cnt):
            p16 = pitv[pl.ds(i * L, L)]
            sel = (p16 >= plo) & (p16 < plo + 16)
            pos = cnt + plsc.cumsum(jnp.where(sel, 1, 0)) - 1
            plsc.store_scatter(wl, [pos], i * L + iota, mask=sel)
            return cnt + plsc.all_reduce_population_count(sel)[0]
        cnt = lax.fori_loop(0, N // L, route, 0)
        ngrp = (cnt + (L - 1)) // L

        for ch in range(NCH):
            c0 = ch * TT
            cw = TT

            def zrow(r, _):
                def zcol(j, _):
                    buf[r, pl.ds(j * L, L)] = jnp.zeros((L,), jnp.float32)
                    return 0
                lax.fori_loop(0, BUFC // L, zcol, 0)
                return 0
            lax.fori_loop(0, 48, zrow, 0)

            def grp_body(g, _):
                ids = wl[pl.ds(g * L, L)]
                sfg = plsc.load_gather(sfv, [ids])
                efg = plsc.load_gather(efv, [ids])
                velg = plsc.load_gather(velv, [ids])
                pitg = plsc.load_gather(pitv, [ids])

                s0i = sfg.astype(jnp.int32)
                cfi = efg.astype(jnp.int32)
                ceii = cfi + jnp.where(cfi.astype(jnp.float32) < efg, 1, 0)
                s0g = s0i.astype(jnp.float32)
                e0g = jnp.minimum(ceii.astype(jnp.float32), T_CAP)
                lo16 = jnp.maximum(jnp.maximum(s0i - W, 0), c0)
                hi16 = jnp.minimum(jnp.minimum(ceii + (W + 1), T), c0 + cw)
                row16 = pitg - plo

                for j in range(L):
                    lane = jnp.full((L,), j, jnp.int32)
                    m = iota == j
                    sfb = _take16(sfg, lane)
                    efb = _take16(efg, lane)
                    velb = _take16(velg, lane)
                    s0b = _take16(s0g, lane)
                    e0b = _take16(e0g, lane)
                    loj = jnp.sum(jnp.where(m, lo16, 0))
                    hij = jnp.sum(jnp.where(m, hi16, 0))
                    rowj = jnp.sum(jnp.where(m, row16, 0))
                    ntrip = (jnp.maximum(hij - loj, 0) + (L - 1)) // L

                    def frame_body(kk, _):
                        t0 = loj + kk * L
                        tvi = t0 + iota
                        tvf = tvi.astype(jnp.float32)
                        valid = tvi < hij
                        zon = (tvf - sfb) * INV_SIG
                        on = jnp.exp(-0.5 * zon * zon) * velb
                        box = jnp.where((tvf >= s0b) & (tvf < e0b), 1.0, 0.0)
                        zof = (tvf - efb) * INV_SIG
                        g10 = jnp.where(tvf >= efb,
                                        jnp.exp(-0.5 * zof * zof) * 0.1, 0.0)
                        sus = box + g10
                        zero = jnp.zeros((L,), jnp.float32)
                        on = jnp.where(valid, on, zero)
                        sus = jnp.where(valid, sus, zero)
                        velc = sus * velb
                        col = t0 - c0
                        plsc.addupdate(buf.at[rowj, pl.ds(col, L)], on)
                        plsc.addupdate(buf.at[rowj + 16, pl.ds(col, L)], sus)
                        plsc.addupdate(buf.at[rowj + 32, pl.ds(col, L)], velc)
                        return 0

                    lax.fori_loop(0, ntrip, frame_body, 0)
                return 0
            lax.fori_loop(0, ngrp, grp_body, 0)

            def crow(r, _):
                def ccol(j, _):
                    v = buf[r, pl.ds(j * L, L)]
                    buf[r, pl.ds(j * L, L)] = jnp.clip(v, 0.0, 1.0)
                    return 0
                lax.fori_loop(0, BUFC // L, ccol, 0)
                return 0
            lax.fori_loop(0, 48, crow, 0)

            pltpu.sync_copy(buf.at[pl.ds(0, 16), pl.ds(0, cw)],
                            out_hbm.at[b, pl.ds(plo, 16), pl.ds(c0, cw)])
            pltpu.sync_copy(buf.at[pl.ds(16, 16), pl.ds(0, cw)],
                            out_hbm.at[b, pl.ds(P + plo, 16), pl.ds(c0, cw)])
            pltpu.sync_copy(buf.at[pl.ds(32, 16), pl.ds(0, cw)],
                            out_hbm.at[b, pl.ds(2 * P + plo, 16), pl.ds(c0, cw)])

    return k(sf, ef, vel, pit)


def kernel(note_start, note_end, note_vel, note_pitch, dur_sec):
    sr = jnp.float32(SR)
    sf = (note_start * sr).astype(jnp.float32)
    ef = (note_end * sr).astype(jnp.float32)
    vel = note_vel.astype(jnp.float32)
    pit = jnp.clip(note_pitch.astype(jnp.int32), 0, P - 1)
    pad = ((0, 0), (0, L))
    sf = jnp.pad(sf, pad, constant_values=-1e6)
    ef = jnp.pad(ef, pad, constant_values=-1e6)
    vel = jnp.pad(vel, pad, constant_values=0.0)
    pit = jnp.pad(pit, pad, constant_values=-1)
    return _render_sc(sf, ef, vel, pit)[:, :, :T]


# speedup vs baseline: 1.5203x; 1.5099x over previous
import functools

import jax
import jax.numpy as jnp
from jax import lax
from jax.experimental import pallas as pl
from jax.experimental.pallas import tpu as pltpu
from jax.experimental.pallas import tpu_sc as plsc

SR = 137.8
P = 128
B = 4
N = 512
T = 4134
SIGMA = 3.6 / 1000.0 * SR
INV_SIG = 1.0 / SIGMA
W = 6
T_CAP = 4134.0
TPAD = 4160

NC = 2
NS = 16
L = 16
NPG = 8
NP = N + L

NCH = 4
TT = 1040
BUFC = TT + L


def _take16(x, idx):
    dnums = lax.GatherDimensionNumbers(
        offset_dims=(), collapsed_slice_dims=(0,), start_index_map=(0,))
    return lax.gather(x, idx[:, None], dnums, (1,),
                      mode=lax.GatherScatterMode.PROMISE_IN_BOUNDS)


def _render_sc(sf, ef, vel, pit):
    mesh = plsc.VectorSubcoreMesh(core_axis_name="c", subcore_axis_name="s")

    @functools.partial(
        pl.kernel,
        mesh=mesh,
        out_type=jax.ShapeDtypeStruct((B, 3 * P, TPAD), jnp.float32),
        scratch_types=[
            pltpu.VMEM((NP,), jnp.float32),
            pltpu.VMEM((NP,), jnp.float32),
            pltpu.VMEM((NP,), jnp.float32),
            pltpu.VMEM((NP,), jnp.int32),
            pltpu.VMEM((NP,), jnp.int32),
            pltpu.VMEM((48, BUFC), jnp.float32),
        ],
        compiler_params=pltpu.CompilerParams(use_tc_tiling_on_sc=False,
                                             needs_layout_passes=False),
    )
    def k(sf_hbm, ef_hbm, vel_hbm, pit_hbm, out_hbm, sfv, efv, velv, pitv, wl, buf):
        cid = lax.axis_index("c")
        sid = lax.axis_index("s")
        wid = sid * NC + cid
        b = wid // NPG
        pg = wid % NPG
        plo = pg * 16

        pltpu.sync_copy(sf_hbm.at[b], sfv)
        pltpu.sync_copy(ef_hbm.at[b], efv)
        pltpu.sync_copy(vel_hbm.at[b], velv)
        pltpu.sync_copy(pit_hbm.at[b], pitv)

        iota = lax.broadcasted_iota(jnp.int32, (L,), 0)

        def init_wl(i, _):
            wl[pl.ds(i * L, L)] = jnp.full((L,), N, jnp.int32)
            return 0
        lax.fori_loop(0, NP // L, init_wl, 0)

        def route(i, cnt):
            p16 = pitv[pl.ds(i * L, L)]
            sel = (p16 >= plo) & (p16 < plo + 16)
            pos = cnt + plsc.cumsum(jnp.where(sel, 1, 0)) - 1
            plsc.store_scatter(wl, [pos], i * L + iota, mask=sel)
            return cnt + plsc.all_reduce_population_count(sel)[0]
        cnt = lax.fori_loop(0, N // L, route, 0)
        ngrp = (cnt + (L - 1)) // L

        def chunk_body(ch, _):
            c0 = pl.multiple_of(ch * TT, TT)

            zv = jnp.zeros((L,), jnp.float32)
            def zrow(r, _):
                def zcol(j, _):
                    for u in range(11):
                        buf[r, pl.ds((j * 11 + u) * L, L)] = zv
                    return 0
                lax.fori_loop(0, 6, zcol, 0)
                return 0
            lax.fori_loop(0, 48, zrow, 0)

            def grp_body(g, _):
                ids = wl[pl.ds(g * L, L)]
                sfg = plsc.load_gather(sfv, [ids])
                efg = plsc.load_gather(efv, [ids])
                velg = plsc.load_gather(velv, [ids])
                pitg = plsc.load_gather(pitv, [ids])

                s0i = sfg.astype(jnp.int32)
                cfi = efg.astype(jnp.int32)
                ceii = cfi + jnp.where(cfi.astype(jnp.float32) < efg, 1, 0)
                s0g = s0i.astype(jnp.float32)
                e0g = jnp.minimum(ceii.astype(jnp.float32), T_CAP)
                lo16 = jnp.maximum(jnp.maximum(s0i - W, 0), c0)
                hi16 = jnp.minimum(jnp.minimum(ceii + (W + 1), T), c0 + TT)
                row16 = pitg - plo

                def lane_body(j, _):
                    lane = jnp.zeros((L,), jnp.int32) + j
                    m = iota == j
                    sfb = _take16(sfg, lane)
                    efb = _take16(efg, lane)
                    velb = _take16(velg, lane)
                    s0b = _take16(s0g, lane)
                    e0b = _take16(e0g, lane)
                    loj = jnp.sum(jnp.where(m, lo16, 0))
                    hij = jnp.sum(jnp.where(m, hi16, 0))
                    rowj = jnp.sum(jnp.where(m, row16, 0))
                    ntrip = (jnp.maximum(hij - loj, 0) + (L - 1)) // L

                    def frame_body(kk, _):
                        t0 = loj + kk * L
                        tvi = t0 + iota
                        tvf = tvi.astype(jnp.float32)
                        valid = tvi < hij
                        zon = (tvf - sfb) * INV_SIG
                        on = jnp.exp(-0.5 * zon * zon) * velb
                        box = jnp.where((tvf >= s0b) & (tvf < e0b), 1.0, 0.0)
                        zof = (tvf - efb) * INV_SIG
                        g10 = jnp.where(tvf >= efb,
                                        jnp.exp(-0.5 * zof * zof) * 0.1, 0.0)
                        sus = box + g10
                        zero = jnp.zeros((L,), jnp.float32)
                        on = jnp.where(valid, on, zero)
                        sus = jnp.where(valid, sus, zero)
                        velc = sus * velb
                        col = t0 - c0
                        plsc.addupdate(buf.at[rowj, pl.ds(col, L)], on)
                        plsc.addupdate(buf.at[rowj + 16, pl.ds(col, L)], sus)
                        plsc.addupdate(buf.at[rowj + 32, pl.ds(col, L)], velc)
                        return 0

                    lax.fori_loop(0, ntrip, frame_body, 0)
                    return 0
                lax.fori_loop(0, L, lane_body, 0)
                return 0
            lax.fori_loop(0, ngrp, grp_body, 0)

            def crow(r, _):
                def ccol(j, _):
                    for u in range(11):
                        sl = pl.ds((j * 11 + u) * L, L)
                        buf[r, sl] = jnp.clip(buf[r, sl], 0.0, 1.0)
                    return 0
                lax.fori_loop(0, 6, ccol, 0)
                return 0
            lax.fori_loop(0, 48, crow, 0)

            pltpu.sync_copy(buf.at[pl.ds(0, 16), pl.ds(0, TT)],
                            out_hbm.at[b, pl.ds(plo, 16), pl.ds(c0, TT)])
            pltpu.sync_copy(buf.at[pl.ds(16, 16), pl.ds(0, TT)],
                            out_hbm.at[b, pl.ds(P + plo, 16), pl.ds(c0, TT)])
            pltpu.sync_copy(buf.at[pl.ds(32, 16), pl.ds(0, TT)],
                            out_hbm.at[b, pl.ds(2 * P + plo, 16), pl.ds(c0, TT)])
            return 0
        lax.fori_loop(0, NCH, chunk_body, 0)

    return k(sf, ef, vel, pit)


def kernel(note_start, note_end, note_vel, note_pitch, dur_sec):
    sr = jnp.float32(SR)
    sf = (note_start * sr).astype(jnp.float32)
    ef = (note_end * sr).astype(jnp.float32)
    vel = note_vel.astype(jnp.float32)
    pit = jnp.clip(note_pitch.astype(jnp.int32), 0, P - 1)
    pad = ((0, 0), (0, L))
    sf = jnp.pad(sf, pad, constant_values=-1e6)
    ef = jnp.pad(ef, pad, constant_values=-1e6)
    vel = jnp.pad(vel, pad, constant_values=0.0)
    pit = jnp.pad(pit, pad, constant_values=-1)
    return _render_sc(sf, ef, vel, pit)[:, :, :T]


# speedup vs baseline: 1.6335x; 1.0744x over previous
import functools

import jax
import jax.numpy as jnp
from jax import lax
from jax.experimental import pallas as pl
from jax.experimental.pallas import tpu as pltpu
from jax.experimental.pallas import tpu_sc as plsc

SR = 137.8
P = 128
B = 4
N = 512
T = 4134
SIGMA = 3.6 / 1000.0 * SR
INV_SIG = 1.0 / SIGMA
W = 6
T_CAP = 4134.0

NC = 2
L = 16
NPG = 8
NP = N + L
NSUB = 4
ROWS = 12
BUFR = ROWS + 1
CSTEP = (T + 16 * 7 - 1) // (16 * 7)


def _take16(x, idx):
    dnums = lax.GatherDimensionNumbers(
        offset_dims=(), collapsed_slice_dims=(0,), start_index_map=(0,))
    return lax.gather(x, idx[:, None], dnums, (1,),
                      mode=lax.GatherScatterMode.PROMISE_IN_BOUNDS)


def _render_sc(sf, ef, vel, pit):
    mesh = plsc.VectorSubcoreMesh(core_axis_name="c", subcore_axis_name="s")

    @functools.partial(
        pl.kernel,
        mesh=mesh,
        out_type=jax.ShapeDtypeStruct((B, 3 * P, T), jnp.float32),
        scratch_types=[
            pltpu.VMEM((NP,), jnp.float32),
            pltpu.VMEM((NP,), jnp.float32),
            pltpu.VMEM((NP,), jnp.float32),
            pltpu.VMEM((NP,), jnp.int32),
            pltpu.VMEM((NSUB, NP), jnp.int32),
            pltpu.VMEM((BUFR, T), jnp.float32),
            pltpu.VMEM((BUFR, T), jnp.float32),
            pltpu.SemaphoreType.DMA,
            pltpu.SemaphoreType.DMA,
        ],
        compiler_params=pltpu.CompilerParams(use_tc_tiling_on_sc=False,
                                             needs_layout_passes=False),
    )
    def k(sf_hbm, ef_hbm, vel_hbm, pit_hbm, out_hbm,
          sfv, efv, velv, pitv, wl, bufa, bufb, sema, semb):
        cid = lax.axis_index("c")
        sid = lax.axis_index("s")
        wid = sid * NC + cid
        b = wid // NPG
        pg = wid % NPG
        plo = pg * 16

        pltpu.sync_copy(sf_hbm.at[b], sfv)
        pltpu.sync_copy(ef_hbm.at[b], efv)
        pltpu.sync_copy(vel_hbm.at[b], velv)
        pltpu.sync_copy(pit_hbm.at[b], pitv)

        iota = lax.broadcasted_iota(jnp.int32, (L,), 0)
        zv = jnp.zeros((L,), jnp.float32)

        def init_wl(i, _):
            for sub in range(NSUB):
                wl[sub, pl.ds(i * L, L)] = jnp.full((L,), N, jnp.int32)
            return 0
        lax.fori_loop(0, NP // L, init_wl, 0)

        ngrps = []
        for sub in range(NSUB):
            slo = plo + sub * 4

            def route(i, cnt, slo=slo, sub=sub):
                p16 = pitv[pl.ds(i * L, L)]
                sel = (p16 >= slo) & (p16 < slo + 4)
                pos = cnt + plsc.cumsum(jnp.where(sel, 1, 0)) - 1
                plsc.store_scatter(wl.at[sub], [pos], i * L + iota, mask=sel)
                return cnt + plsc.all_reduce_population_count(sel)[0]
            cnt = lax.fori_loop(0, N // L, route, 0)
            ngrps.append((cnt + (L - 1)) // L)

        bufs = (bufa, bufb)
        sems = (sema, semb)
        descs = {}
        for sub in range(NSUB):
            buf = bufs[sub % 2]
            sem = sems[sub % 2]
            slo = plo + sub * 4

            if sub >= 2:
                for cp in descs[sub - 2]:
                    cp.wait()

            def zrow(r, _, buf=buf):
                def zcol(j, _):
                    for u in range(7):
                        buf[r, pl.ds((j * 7 + u) * L, L)] = zv
                    return 0
                lax.fori_loop(0, CSTEP, zcol, 0)
                return 0
            lax.fori_loop(0, ROWS, zrow, 0)

            def grp_body(g, _, slo=slo, sub=sub, buf=buf):
                ids = wl[sub, pl.ds(g * L, L)]
                sfg = plsc.load_gather(sfv, [ids])
                efg = plsc.load_gather(efv, [ids])
                velg = plsc.load_gather(velv, [ids])
                pitg = plsc.load_gather(pitv, [ids])

                s0i = sfg.astype(jnp.int32)
                cfi = efg.astype(jnp.int32)
                ceii = cfi + jnp.where(cfi.astype(jnp.float32) < efg, 1, 0)
                s0g = s0i.astype(jnp.float32)
                e0g = jnp.minimum(ceii.astype(jnp.float32), T_CAP)
                lo16 = jnp.maximum(s0i - W, 0)
                hi16 = jnp.minimum(ceii + (W + 1), T)
                row16 = pitg - slo

                def lane_body(j, _):
                    lane = jnp.zeros((L,), jnp.int32) + j
                    m = iota == j
                    sfb = _take16(sfg, lane)
                    efb = _take16(efg, lane)
                    velb = _take16(velg, lane)
                    s0b = _take16(s0g, lane)
                    e0b = _take16(e0g, lane)
                    loj = jnp.sum(jnp.where(m, lo16, 0))
                    hij = jnp.sum(jnp.where(m, hi16, 0))
                    rowj = jnp.sum(jnp.where(m, row16, 0))
                    ntrip = (jnp.maximum(hij - loj, 0) + (L - 1)) // L

                    def frame_body(kk, _):
                        t0 = loj + kk * L
                        tvi = t0 + iota
                        tvf = tvi.astype(jnp.float32)
                        valid = tvi < hij
                        zon = (tvf - sfb) * INV_SIG
                        on = jnp.exp(-0.5 * zon * zon) * velb
                        box = jnp.where((tvf >= s0b) & (tvf < e0b), 1.0, 0.0)
                        zof = (tvf - efb) * INV_SIG
                        g10 = jnp.where(tvf >= efb,
                                        jnp.exp(-0.5 * zof * zof) * 0.1, 0.0)
                        sus = box + g10
                        on = jnp.where(valid, on, zv)
                        sus = jnp.where(valid, sus, zv)
                        velc = sus * velb
                        plsc.addupdate(buf.at[rowj, pl.ds(t0, L)], on)
                        plsc.addupdate(buf.at[rowj + 4, pl.ds(t0, L)], sus)
                        plsc.addupdate(buf.at[rowj + 8, pl.ds(t0, L)], velc)
                        return 0

                    lax.fori_loop(0, ntrip, frame_body, 0)
                    return 0
                lax.fori_loop(0, L, lane_body, 0)
                return 0
            lax.fori_loop(0, ngrps[sub], grp_body, 0)

            def crow(r, _, buf=buf):
                def ccol(j, _):
                    for u in range(7):
                        sl = pl.ds((j * 7 + u) * L, L)
                        buf[r, sl] = jnp.clip(buf[r, sl], 0.0, 1.0)
                    return 0
                lax.fori_loop(0, CSTEP, ccol, 0)
                return 0
            lax.fori_loop(0, ROWS, crow, 0)

            cps = [
                pltpu.make_async_copy(buf.at[pl.ds(0, 4)],
                                      out_hbm.at[b, pl.ds(slo, 4)], sem),
                pltpu.make_async_copy(buf.at[pl.ds(4, 4)],
                                      out_hbm.at[b, pl.ds(P + slo, 4)], sem),
                pltpu.make_async_copy(buf.at[pl.ds(8, 4)],
                                      out_hbm.at[b, pl.ds(2 * P + slo, 4)], sem),
            ]
            for cp in cps:
                cp.start()
            descs[sub] = cps

        for sub in (NSUB - 2, NSUB - 1):
            for cp in descs[sub]:
                cp.wait()

    return k(sf, ef, vel, pit)


def kernel(note_start, note_end, note_vel, note_pitch, dur_sec):
    sr = jnp.float32(SR)
    sf = (note_start * sr).astype(jnp.float32)
    ef = (note_end * sr).astype(jnp.float32)
    vel = note_vel.astype(jnp.float32)
    pit = jnp.clip(note_pitch.astype(jnp.int32), 0, P - 1)
    pad = ((0, 0), (0, L))
    sf = jnp.pad(sf, pad, constant_values=-1e6)
    ef = jnp.pad(ef, pad, constant_values=-1e6)
    vel = jnp.pad(vel, pad, constant_values=0.0)
    pit = jnp.pad(pit, pad, constant_values=-1)
    return _render_sc(sf, ef, vel, pit)


# speedup vs baseline: 2.0813x; 1.2742x over previous
import functools

import jax
import jax.numpy as jnp
from jax import lax
from jax.experimental import pallas as pl
from jax.experimental.pallas import tpu as pltpu
from jax.experimental.pallas import tpu_sc as plsc

SR = 137.8
P = 128
B = 4
N = 512
T = 4134
SIGMA = 3.6 / 1000.0 * SR
INV_SIG = 1.0 / SIGMA
W = 6
T_CAP = 4134.0

NC = 2
L = 16
NPG = 8
NP = N + L
NSUB = 4
ROWS = 12
BUFR = ROWS + 1


def _take16(x, idx):
    dnums = lax.GatherDimensionNumbers(
        offset_dims=(), collapsed_slice_dims=(0,), start_index_map=(0,))
    return lax.gather(x, idx[:, None], dnums, (1,),
                      mode=lax.GatherScatterMode.PROMISE_IN_BOUNDS)


def _render_sc(sf, ef, vel, pit, zrows):
    mesh = plsc.VectorSubcoreMesh(core_axis_name="c", subcore_axis_name="s")

    @functools.partial(
        pl.kernel,
        mesh=mesh,
        out_type=jax.ShapeDtypeStruct((B, 3 * P, T), jnp.float32),
        scratch_types=[
            pltpu.VMEM((NP,), jnp.float32),
            pltpu.VMEM((NP,), jnp.float32),
            pltpu.VMEM((NP,), jnp.float32),
            pltpu.VMEM((NP,), jnp.int32),
            pltpu.VMEM((NSUB, NP), jnp.int32),
            pltpu.VMEM((BUFR, T), jnp.float32),
            pltpu.VMEM((BUFR, T), jnp.float32),
            pltpu.SemaphoreType.DMA,
            pltpu.SemaphoreType.DMA,
            pltpu.SemaphoreType.DMA,
            pltpu.SemaphoreType.DMA,
        ],
        compiler_params=pltpu.CompilerParams(use_tc_tiling_on_sc=False,
                                             needs_layout_passes=False),
    )
    def k(sf_hbm, ef_hbm, vel_hbm, pit_hbm, z_hbm, out_hbm,
          sfv, efv, velv, pitv, wl, bufa, bufb, osema, osemb, zsema, zsemb):
        cid = lax.axis_index("c")
        sid = lax.axis_index("s")
        wid = sid * NC + cid
        b = wid // NPG
        pg = wid % NPG
        plo = pg * 16

        pltpu.sync_copy(sf_hbm.at[b], sfv)
        pltpu.sync_copy(ef_hbm.at[b], efv)
        pltpu.sync_copy(vel_hbm.at[b], velv)
        pltpu.sync_copy(pit_hbm.at[b], pitv)

        iota = lax.broadcasted_iota(jnp.int32, (L,), 0)
        zv = jnp.zeros((L,), jnp.float32)
        bufs = (bufa, bufb)
        osems = (osema, osemb)
        zsems = (zsema, zsemb)

        zdesc = {0: pltpu.make_async_copy(z_hbm, bufa, zsema)}
        zdesc[0].start()

        def init_wl(i, _):
            for sub in range(NSUB):
                wl[sub, pl.ds(i * L, L)] = jnp.full((L,), N, jnp.int32)
            return 0
        lax.fori_loop(0, NP // L, init_wl, 0)

        ngrps = []
        for sub in range(NSUB):
            slo = plo + sub * 4

            def route(i, cnt, slo=slo, sub=sub):
                p16 = pitv[pl.ds(i * L, L)]
                sel = (p16 >= slo) & (p16 < slo + 4)
                pos = cnt + plsc.cumsum(jnp.where(sel, 1, 0)) - 1
                plsc.store_scatter(wl.at[sub], [pos], i * L + iota, mask=sel)
                return cnt + plsc.all_reduce_population_count(sel)[0]
            cnt = lax.fori_loop(0, N // L, route, 0)
            ngrps.append((cnt + (L - 1)) // L)

        odescs = {}
        for sub in range(NSUB):
            buf = bufs[sub % 2]
            slo = plo + sub * 4

            zdesc[sub].wait()

            def grp_body(g, ext, slo=slo, sub=sub, buf=buf):
                ids = wl[sub, pl.ds(g * L, L)]
                sfg = plsc.load_gather(sfv, [ids])
                efg = plsc.load_gather(efv, [ids])
                velg = plsc.load_gather(velv, [ids])
                pitg = plsc.load_gather(pitv, [ids])

                s0i = sfg.astype(jnp.int32)
                cfi = efg.astype(jnp.int32)
                ceii = cfi + jnp.where(cfi.astype(jnp.float32) < efg, 1, 0)
                s0g = s0i.astype(jnp.float32)
                e0g = jnp.minimum(ceii.astype(jnp.float32), T_CAP)
                lo16 = jnp.maximum(s0i - W, 0)
                hi16 = jnp.minimum(ceii + (W + 1), T)
                row16 = pitg - slo

                live = hi16 > lo16
                next_ext = []
                for r in range(4):
                    mr = live & (row16 == r)
                    rmin = jnp.min(jnp.where(mr, lo16, T))
                    rmax = jnp.max(jnp.where(mr, hi16, 0))
                    next_ext.append(jnp.minimum(ext[2 * r], rmin))
                    next_ext.append(jnp.maximum(ext[2 * r + 1], rmax))

                def lane_body(j, _):
                    lane = jnp.zeros((L,), jnp.int32) + j
                    m = iota == j
                    sfb = _take16(sfg, lane)
                    efb = _take16(efg, lane)
                    velb = _take16(velg, lane)
                    s0b = _take16(s0g, lane)
                    e0b = _take16(e0g, lane)
                    loj = jnp.sum(jnp.where(m, lo16, 0))
                    hij = jnp.sum(jnp.where(m, hi16, 0))
                    rowj = jnp.sum(jnp.where(m, row16, 0))
                    ntrip = (jnp.maximum(hij - loj, 0) + (L - 1)) // L

                    def frame_body(kk, _):
                        t0 = loj + kk * L
                        tvi = t0 + iota
                        tvf = tvi.astype(jnp.float32)
                        valid = tvi < hij
                        zon = (tvf - sfb) * INV_SIG
                        on = jnp.exp(-0.5 * zon * zon) * velb
                        box = jnp.where((tvf >= s0b) & (tvf < e0b), 1.0, 0.0)
                        zof = (tvf - efb) * INV_SIG
                        g10 = jnp.where(tvf >= efb,
                                        jnp.exp(-0.5 * zof * zof) * 0.1, 0.0)
                        sus = box + g10
                        on = jnp.where(valid, on, zv)
                        sus = jnp.where(valid, sus, zv)
                        velc = sus * velb
                        plsc.addupdate(buf.at[rowj, pl.ds(t0, L)], on)
                        plsc.addupdate(buf.at[rowj + 4, pl.ds(t0, L)], sus)
                        plsc.addupdate(buf.at[rowj + 8, pl.ds(t0, L)], velc)
                        return 0

                    lax.fori_loop(0, ntrip, frame_body, 0)
                    return 0
                lax.fori_loop(0, L, lane_body, 0)
                return tuple(next_ext)

            ext0 = []
            for r in range(4):
                ext0.extend([jnp.int32(T), jnp.int32(0)])
            ext = lax.fori_loop(0, ngrps[sub], grp_body, tuple(ext0))

            for r in range(4):
                base = ext[2 * r] & ~(L - 1)
                n16 = jnp.maximum(ext[2 * r + 1] - base + (L - 1), 0) // L

                def crow(j, _, r=r, base=base, buf=buf):
                    sl = pl.ds(base + j * L, L)
                    buf[r, sl] = jnp.clip(buf[r, sl], 0.0, 1.0)
                    sl = pl.ds(base + j * L, L)
                    buf[r + 4, sl] = jnp.clip(buf[r + 4, sl], 0.0, 1.0)
                    sl = pl.ds(base + j * L, L)
                    buf[r + 8, sl] = jnp.clip(buf[r + 8, sl], 0.0, 1.0)
                    return 0
                lax.fori_loop(0, n16, crow, 0)

            cps = [
                pltpu.make_async_copy(buf.at[pl.ds(0, 4)],
                                      out_hbm.at[b, pl.ds(slo, 4)],
                                      osems[sub % 2]),
                pltpu.make_async_copy(buf.at[pl.ds(4, 4)],
                                      out_hbm.at[b, pl.ds(P + slo, 4)],
                                      osems[sub % 2]),
                pltpu.make_async_copy(buf.at[pl.ds(8, 4)],
                                      out_hbm.at[b, pl.ds(2 * P + slo, 4)],
                                      osems[sub % 2]),
            ]
            for cp in cps:
                cp.start()
            odescs[sub] = cps

            if sub + 1 < NSUB:
                if sub >= 1:
                    for cp in odescs[sub - 1]:
                        cp.wait()
                zdesc[sub + 1] = pltpu.make_async_copy(
                    z_hbm, bufs[(sub + 1) % 2], zsems[(sub + 1) % 2])
                zdesc[sub + 1].start()

        for sub in (NSUB - 2, NSUB - 1):
            for cp in odescs[sub]:
                cp.wait()

    return k(sf, ef, vel, pit, zrows)


def kernel(note_start, note_end, note_vel, note_pitch, dur_sec):
    sr = jnp.float32(SR)
    sf = (note_start * sr).astype(jnp.float32)
    ef = (note_end * sr).astype(jnp.float32)
    vel = note_vel.astype(jnp.float32)
    pit = jnp.clip(note_pitch.astype(jnp.int32), 0, P - 1)
    pad = ((0, 0), (0, L))
    sf = jnp.pad(sf, pad, constant_values=-1e6)
    ef = jnp.pad(ef, pad, constant_values=-1e6)
    vel = jnp.pad(vel, pad, constant_values=0.0)
    pit = jnp.pad(pit, pad, constant_values=-1)
    zrows = jnp.zeros((BUFR, T), jnp.float32)
    return _render_sc(sf, ef, vel, pit, zrows)


# speedup vs baseline: 2.3525x; 1.1303x over previous
import functools

import jax
import jax.numpy as jnp
from jax import lax
from jax.experimental import pallas as pl
from jax.experimental.pallas import tpu as pltpu
from jax.experimental.pallas import tpu_sc as plsc

SR = 137.8
P = 128
B = 4
N = 512
T = 4134
SIGMA = 3.6 / 1000.0 * SR
INV_SIG = 1.0 / SIGMA
W = 6
T_CAP = 4134.0

NC = 2
L = 16
NPG = 8
NP = N + L
NSUB = 2
SUBP = 8


def _take16(x, idx):
    dnums = lax.GatherDimensionNumbers(
        offset_dims=(), collapsed_slice_dims=(0,), start_index_map=(0,))
    return lax.gather(x, idx[:, None], dnums, (1,),
                      mode=lax.GatherScatterMode.PROMISE_IN_BOUNDS)


def _render_sc(sf, ef, vel, pit, zrows):
    mesh = plsc.VectorSubcoreMesh(core_axis_name="c", subcore_axis_name="s")

    @functools.partial(
        pl.kernel,
        mesh=mesh,
        out_type=jax.ShapeDtypeStruct((B, 3 * P, T), jnp.float32),
        scratch_types=[
            pltpu.VMEM((NP,), jnp.float32),
            pltpu.VMEM((NP,), jnp.float32),
            pltpu.VMEM((NP,), jnp.float32),
            pltpu.VMEM((NP,), jnp.int32),
            pltpu.VMEM((NSUB * NP,), jnp.int32),
            pltpu.VMEM((SUBP, T), jnp.float32),
            pltpu.VMEM((SUBP, T), jnp.float32),
            pltpu.VMEM((SUBP, T), jnp.float32),
            pltpu.SemaphoreType.DMA,
            pltpu.SemaphoreType.DMA,
        ],
        compiler_params=pltpu.CompilerParams(use_tc_tiling_on_sc=True,
                                             needs_layout_passes=False),
    )
    def k(sf_hbm, ef_hbm, vel_hbm, pit_hbm, z_hbm, out_hbm,
          sfv, efv, velv, pitv, wl, bon, bsus, bvel, osem, zsem):
        cid = lax.axis_index("c")
        sid = lax.axis_index("s")
        wid = sid * NC + cid
        b = wid // NPG
        pg = wid % NPG
        plo = pg * 16

        bufs = (bon, bsus, bvel)

        zdescs = [pltpu.make_async_copy(z_hbm, bf, zsem) for bf in bufs]
        for cp in zdescs:
            cp.start()

        pltpu.sync_copy(sf_hbm.at[b], sfv)
        pltpu.sync_copy(ef_hbm.at[b], efv)
        pltpu.sync_copy(vel_hbm.at[b], velv)
        pltpu.sync_copy(pit_hbm.at[b], pitv)

        iota = lax.broadcasted_iota(jnp.int32, (L,), 0)

        def init_wl(i, _):
            wl[pl.ds(i * L, L)] = jnp.full((L,), N, jnp.int32)
            return 0
        lax.fori_loop(0, NSUB * NP // L, init_wl, 0)

        ngrps = []
        for sub in range(NSUB):
            slo = plo + sub * SUBP

            def route(i, cnt, slo=slo, sub=sub):
                p16 = pitv[pl.ds(i * L, L)]
                sel = (p16 >= slo) & (p16 < slo + SUBP)
                pos = sub * NP + cnt + plsc.cumsum(jnp.where(sel, 1, 0)) - 1
                plsc.store_scatter(wl, [pos], i * L + iota, mask=sel)
                return cnt + plsc.all_reduce_population_count(sel)[0]
            cnt = lax.fori_loop(0, N // L, route, 0)
            ngrps.append((cnt + (L - 1)) // L)

        for sub in range(NSUB):
            slo = plo + sub * SUBP

            for cp in zdescs:
                cp.wait()

            def grp_body(g, ext, slo=slo, sub=sub):
                ids = wl[pl.ds(sub * NP + g * L, L)]
                sfg = plsc.load_gather(sfv, [ids])
                efg = plsc.load_gather(efv, [ids])
                velg = plsc.load_gather(velv, [ids])
                pitg = plsc.load_gather(pitv, [ids])

                s0i = sfg.astype(jnp.int32)
                cfi = efg.astype(jnp.int32)
                ceii = cfi + jnp.where(cfi.astype(jnp.float32) < efg, 1, 0)
                s0g = s0i.astype(jnp.float32)
                e0g = jnp.minimum(ceii.astype(jnp.float32), T_CAP)
                lo16 = jnp.maximum(s0i - W, 0) & ~(L - 1)
                hi16 = jnp.minimum(ceii + (W + 1), T)
                row16 = pitg - slo

                live = hi16 > lo16
                next_ext = []
                for r in range(SUBP):
                    mr = live & (row16 == r)
                    rmin = jnp.min(jnp.where(mr, lo16, T))
                    rmax = jnp.max(jnp.where(mr, hi16, 0))
                    next_ext.append(jnp.minimum(ext[2 * r], rmin))
                    next_ext.append(jnp.maximum(ext[2 * r + 1], rmax))

                def lane_body(j, _):
                    lane = jnp.zeros((L,), jnp.int32) + j
                    m = iota == j
                    sfb = _take16(sfg, lane)
                    efb = _take16(efg, lane)
                    velb = _take16(velg, lane)
                    s0b = _take16(s0g, lane)
                    e0b = _take16(e0g, lane)
                    loj = jnp.sum(jnp.where(m, lo16, 0))
                    hij = jnp.sum(jnp.where(m, hi16, 0))
                    rowj = jnp.sum(jnp.where(m, row16, 0))
                    ntrip = (jnp.maximum(hij - loj, 0) + (L - 1)) // L

                    def frame_body(kk, _):
                        t0 = loj + kk * L
                        tvi = t0 + iota
                        tvf = tvi.astype(jnp.float32)
                        zon = (tvf - sfb) * INV_SIG
                        on = jnp.exp(-0.5 * zon * zon) * velb
                        box = jnp.where((tvf >= s0b) & (tvf < e0b), 1.0, 0.0)
                        zof = (tvf - efb) * INV_SIG
                        g10 = jnp.where(tvf >= efb,
                                        jnp.exp(-0.5 * zof * zof) * 0.1, 0.0)
                        sus = box + g10
                        velc = sus * velb
                        plsc.addupdate(bon.at[rowj, pl.ds(t0, L)], on)
                        plsc.addupdate(bsus.at[rowj, pl.ds(t0, L)], sus)
                        plsc.addupdate(bvel.at[rowj, pl.ds(t0, L)], velc)
                        return 0

                    lax.fori_loop(0, ntrip, frame_body, 0)
                    return 0
                lax.fori_loop(0, L, lane_body, 0)
                return tuple(next_ext)

            ext0 = []
            for r in range(SUBP):
                ext0.extend([jnp.int32(T), jnp.int32(0)])
            ext = lax.fori_loop(0, ngrps[sub], grp_body, tuple(ext0))

            for r in range(SUBP):
                base = ext[2 * r]
                n16 = jnp.maximum(ext[2 * r + 1] - base + (L - 1), 0) // L

                def crow(j, _, r=r, base=base):
                    sl = pl.ds(base + j * L, L)
                    bon[r, sl] = jnp.clip(bon[r, sl], 0.0, 1.0)
                    sl = pl.ds(base + j * L, L)
                    bsus[r, sl] = jnp.clip(bsus[r, sl], 0.0, 1.0)
                    sl = pl.ds(base + j * L, L)
                    bvel[r, sl] = jnp.clip(bvel[r, sl], 0.0, 1.0)
                    return 0
                lax.fori_loop(0, n16, crow, 0)

            odescs = [
                pltpu.make_async_copy(bon, out_hbm.at[b, pl.ds(slo, SUBP)],
                                      osem),
                pltpu.make_async_copy(bsus, out_hbm.at[b, pl.ds(P + slo, SUBP)],
                                      osem),
                pltpu.make_async_copy(bvel, out_hbm.at[b, pl.ds(2 * P + slo, SUBP)],
                                      osem),
            ]
            for cp in odescs:
                cp.start()
            for cp in odescs:
                cp.wait()
            if sub + 1 < NSUB:
                zdescs = [pltpu.make_async_copy(z_hbm, bf, zsem) for bf in bufs]
                for cp in zdescs:
                    cp.start()

    return k(sf, ef, vel, pit, zrows)


def kernel(note_start, note_end, note_vel, note_pitch, dur_sec):
    sr = jnp.float32(SR)
    sf = (note_start * sr).astype(jnp.float32)
    ef = (note_end * sr).astype(jnp.float32)
    vel = note_vel.astype(jnp.float32)
    pit = jnp.clip(note_pitch.astype(jnp.int32), 0, P - 1)
    pad = ((0, 0), (0, L))
    sf = jnp.pad(sf, pad, constant_values=-1e6)
    ef = jnp.pad(ef, pad, constant_values=-1e6)
    vel = jnp.pad(vel, pad, constant_values=0.0)
    pit = jnp.pad(pit, pad, constant_values=-1)
    zrows = jnp.zeros((SUBP, T), jnp.float32)
    return _render_sc(sf, ef, vel, pit, zrows)


# speedup vs baseline: 2.3632x; 1.0046x over previous
import functools

import jax
import jax.numpy as jnp
from jax import lax
from jax.experimental import pallas as pl
from jax.experimental.pallas import tpu as pltpu
from jax.experimental.pallas import tpu_sc as plsc

SR = 137.8
P = 128
B = 4
N = 512
T = 4134
SIGMA = 3.6 / 1000.0 * SR
INV_SIG = 1.0 / SIGMA
W = 6
T_CAP = 4134.0

NC = 2
L = 16
NPG = 8
NP = N + L
NSUB = 2
SUBP = 8


def _take16(x, idx):
    dnums = lax.GatherDimensionNumbers(
        offset_dims=(), collapsed_slice_dims=(0,), start_index_map=(0,))
    return lax.gather(x, idx[:, None], dnums, (1,),
                      mode=lax.GatherScatterMode.PROMISE_IN_BOUNDS)


def _render_sc(sf, ef, vel, pit, zrows):
    mesh = plsc.VectorSubcoreMesh(core_axis_name="c", subcore_axis_name="s")

    @functools.partial(
        pl.kernel,
        mesh=mesh,
        out_type=jax.ShapeDtypeStruct((B, 3 * P, T), jnp.float32),
        scratch_types=[
            pltpu.VMEM((NP,), jnp.float32),
            pltpu.VMEM((NP,), jnp.float32),
            pltpu.VMEM((NP,), jnp.float32),
            pltpu.VMEM((NP,), jnp.int32),
            pltpu.VMEM((NSUB * NP,), jnp.int32),
            pltpu.VMEM((SUBP, T), jnp.float32),
            pltpu.VMEM((SUBP, T), jnp.float32),
            pltpu.VMEM((SUBP, T), jnp.float32),
            pltpu.SemaphoreType.DMA,
            pltpu.SemaphoreType.DMA,
        ],
        compiler_params=pltpu.CompilerParams(use_tc_tiling_on_sc=True,
                                             needs_layout_passes=False),
    )
    def k(sf_hbm, ef_hbm, vel_hbm, pit_hbm, z_hbm, out_hbm,
          sfv, efv, velv, pitv, wl, bon, bsus, bvel, osem, zsem):
        cid = lax.axis_index("c")
        sid = lax.axis_index("s")
        wid = sid * NC + cid
        b = wid // NPG
        pg = wid % NPG
        plo = pg * 16

        bufs = (bon, bsus, bvel)

        zdescs = [pltpu.make_async_copy(z_hbm, bf, zsem) for bf in bufs]
        for cp in zdescs:
            cp.start()

        pltpu.sync_copy(sf_hbm.at[b], sfv)
        pltpu.sync_copy(ef_hbm.at[b], efv)
        pltpu.sync_copy(vel_hbm.at[b], velv)
        pltpu.sync_copy(pit_hbm.at[b], pitv)

        iota = lax.broadcasted_iota(jnp.int32, (L,), 0)

        ngrps = []
        for sub in range(NSUB):
            slo = plo + sub * SUBP

            def route(i, cnt, slo=slo, sub=sub):
                p16 = pitv[pl.ds(i * L, L)]
                sel = (p16 >= slo) & (p16 < slo + SUBP)
                pos = sub * NP + cnt + plsc.cumsum(jnp.where(sel, 1, 0)) - 1
                plsc.store_scatter(wl, [pos], i * L + iota, mask=sel)
                return cnt + plsc.all_reduce_population_count(sel)[0]
            cnt = lax.fori_loop(0, N // L, route, 0)
            wl[pl.ds(sub * NP + cnt, L)] = jnp.full((L,), N, jnp.int32)
            ngrps.append((cnt + (L - 1)) // L)

        for sub in range(NSUB):
            slo = plo + sub * SUBP

            for cp in zdescs:
                cp.wait()

            def grp_body(g, ext, slo=slo, sub=sub):
                ids = wl[pl.ds(sub * NP + g * L, L)]
                sfg = plsc.load_gather(sfv, [ids])
                efg = plsc.load_gather(efv, [ids])
                velg = plsc.load_gather(velv, [ids])
                pitg = plsc.load_gather(pitv, [ids])

                s0i = sfg.astype(jnp.int32)
                cfi = efg.astype(jnp.int32)
                ceii = cfi + jnp.where(cfi.astype(jnp.float32) < efg, 1, 0)
                s0g = s0i.astype(jnp.float32)
                e0g = jnp.minimum(ceii.astype(jnp.float32), T_CAP)
                lo16 = jnp.maximum(s0i - W, 0) & ~(L - 1)
                hi16 = jnp.minimum(ceii + (W + 1), T)
                row16 = pitg - slo

                live = hi16 > lo16
                next_ext = []
                for r in range(SUBP):
                    mr = live & (row16 == r)
                    rmin = jnp.min(jnp.where(mr, lo16, T))
                    rmax = jnp.max(jnp.where(mr, hi16, 0))
                    next_ext.append(jnp.minimum(ext[2 * r], rmin))
                    next_ext.append(jnp.maximum(ext[2 * r + 1], rmax))

                def lane_body(j, _):
                    lane = jnp.zeros((L,), jnp.int32) + j
                    m = iota == j
                    sfb = _take16(sfg, lane)
                    efb = _take16(efg, lane)
                    velb = _take16(velg, lane)
                    s0b = _take16(s0g, lane)
                    e0b = _take16(e0g, lane)
                    loj = jnp.sum(jnp.where(m, lo16, 0))
                    hij = jnp.sum(jnp.where(m, hi16, 0))
                    rowj = jnp.sum(jnp.where(m, row16, 0))
                    ntrip = (jnp.maximum(hij - loj, 0) + (L - 1)) // L

                    def frame_body(kk, _):
                        t0 = loj + kk * L
                        tvi = t0 + iota
                        tvf = tvi.astype(jnp.float32)
                        zon = (tvf - sfb) * INV_SIG
                        on = jnp.exp(-0.5 * zon * zon) * velb
                        box = jnp.where((tvf >= s0b) & (tvf < e0b), 1.0, 0.0)
                        zof = (tvf - efb) * INV_SIG
                        g10 = jnp.where(tvf >= efb,
                                        jnp.exp(-0.5 * zof * zof) * 0.1, 0.0)
                        sus = box + g10
                        velc = sus * velb
                        plsc.addupdate(bon.at[rowj, pl.ds(t0, L)], on)
                        plsc.addupdate(bsus.at[rowj, pl.ds(t0, L)], sus)
                        plsc.addupdate(bvel.at[rowj, pl.ds(t0, L)], velc)
                        return 0

                    lax.fori_loop(0, ntrip, frame_body, 0)
                    return 0
                lax.fori_loop(0, L, lane_body, 0)
                return tuple(next_ext)

            ext0 = []
            for r in range(SUBP):
                ext0.extend([jnp.int32(T), jnp.int32(0)])
            ext = lax.fori_loop(0, ngrps[sub], grp_body, tuple(ext0))

            for r in range(SUBP):
                base = ext[2 * r]
                n16 = jnp.maximum(ext[2 * r + 1] - base + (L - 1), 0) // L

                def crow(j, _, r=r, base=base):
                    sl = pl.ds(base + j * L, L)
                    bon[r, sl] = jnp.clip(bon[r, sl], 0.0, 1.0)
                    sl = pl.ds(base + j * L, L)
                    bsus[r, sl] = jnp.clip(bsus[r, sl], 0.0, 1.0)
                    sl = pl.ds(base + j * L, L)
                    bvel[r, sl] = jnp.clip(bvel[r, sl], 0.0, 1.0)
                    return 0
                lax.fori_loop(0, n16, crow, 0)

            odescs = [
                pltpu.make_async_copy(bon, out_hbm.at[b, pl.ds(slo, SUBP)],
                                      osem),
                pltpu.make_async_copy(bsus, out_hbm.at[b, pl.ds(P + slo, SUBP)],
                                      osem),
                pltpu.make_async_copy(bvel, out_hbm.at[b, pl.ds(2 * P + slo, SUBP)],
                                      osem),
            ]
            for cp in odescs:
                cp.start()
            if sub + 1 < NSUB:
                zdescs = []
                for ch in range(3):
                    odescs[ch].wait()
                    zcp = pltpu.make_async_copy(z_hbm, bufs[ch], zsem)
                    zcp.start()
                    zdescs.append(zcp)
            else:
                for cp in odescs:
                    cp.wait()

    return k(sf, ef, vel, pit, zrows)


def kernel(note_start, note_end, note_vel, note_pitch, dur_sec):
    sr = jnp.float32(SR)
    sf = (note_start * sr).astype(jnp.float32)
    ef = (note_end * sr).astype(jnp.float32)
    vel = note_vel.astype(jnp.float32)
    pit = jnp.clip(note_pitch.astype(jnp.int32), 0, P - 1)
    pad = ((0, 0), (0, L))
    sf = jnp.pad(sf, pad, constant_values=-1e6)
    ef = jnp.pad(ef, pad, constant_values=-1e6)
    vel = jnp.pad(vel, pad, constant_values=0.0)
    pit = jnp.pad(pit, pad, constant_values=-1)
    zrows = jnp.zeros((SUBP, T), jnp.float32)
    return _render_sc(sf, ef, vel, pit, zrows)
